# baseline (speedup 1.0000x reference)
import jax
import jax.numpy as jnp
from jax import lax
from jax.experimental import pallas as pl
from jax.experimental.pallas import tpu as pltpu

_MARGIN = 0.1
_ALPHA = 0.5
_BETA = 0.5
_C = 32
_N = 1024
_D = 2048
_F = 128
_NF = _N // _F
_S = _N // 2
_NEG_INF = float('-inf')
_POS_INF = float('inf')


def _nt(a, b):
    return lax.dot_general(a, b, (((1,), (1,)), ((), ())),
                           preferred_element_type=jnp.float32)


def _ntb(a, b):
    return lax.dot_general(a, b, (((1,), (1,)), ((), ())),
                           preferred_element_type=jnp.float32
                           ).astype(jnp.bfloat16)


def _block_updates(g, hb_col, hb_row, tc, tr, labels_row):
    v = jnp.where(tc == tr, hb_col + hb_row - g,
                  jnp.bfloat16(_NEG_INF))
    colmax = jnp.max(v, axis=0, keepdims=True).astype(jnp.float32)
    rowmax = jnp.max(v, axis=1, keepdims=True).astype(jnp.float32)
    rcls = jnp.max(jnp.where(tc == labels_row, rowmax, _NEG_INF),
                   axis=0, keepdims=True)
    return colmax, rcls


def _body(fs_ref, tcol_ref, trow_ref, out_ref,
          fall_scr, hbc_scr, hbr_scr, cen_scr, colmax_scr, rcls_scr):
    s = pl.program_id(0)
    fs = fs_ref[...]
    labels_row = lax.broadcasted_iota(jnp.int32, (1, _C), 1)
    ones_row = jnp.ones((1, _D), jnp.float32)

    ff = fs * fs
    hsq_f = 0.5 * _nt(ff, ones_row)
    hsqr_f = 0.5 * _nt(ones_row, ff)
    fall_scr[pl.ds(s * _F, _F), :] = fs.astype(jnp.bfloat16)
    hbc_scr[pl.ds(s * _F, _F), :] = hsq_f.astype(jnp.bfloat16)
    hbr_scr[:, pl.ds(s * _F, _F)] = hsqr_f.astype(jnp.bfloat16)
    tc_f = tcol_ref[pl.ds(s * _F, _F), :]
    onehot_f = (tc_f == labels_row).astype(jnp.float32)
    cen_part = lax.dot_general(onehot_f, fs, (((0,), (0,)), ((), ())),
                               preferred_element_type=jnp.float32)

    @pl.when(s == 0)
    def _cen0():
        cen_scr[...] = cen_part

    @pl.when(s > 0)
    def _cenacc():
        cen_scr[...] += cen_part

    @pl.when(s == _NF // 2 - 1)
    def _phase_a():
        fb0 = fall_scr[pl.ds(0, _S), :]
        hb0 = hbc_scr[pl.ds(0, _S), :]
        hbr0 = hbr_scr[:, pl.ds(0, _S)]
        tc0 = tcol_ref[pl.ds(0, _S), :]
        tr0 = trow_ref[:, pl.ds(0, _S)]
        g00 = _ntb(fb0, fb0)
        cm0, rc0 = _block_updates(g00, hb0, hbr0, tc0, tr0, labels_row)
        colmax_scr[:, pl.ds(0, _S)] = cm0
        rcls_scr[...] = rc0

    @pl.when(s == _NF - 1)
    def _phase_b():
        fb0 = fall_scr[pl.ds(0, _S), :]
        fb1 = fall_scr[pl.ds(_S, _S), :]
        hb1 = hbc_scr[pl.ds(_S, _S), :]
        hbr0 = hbr_scr[:, pl.ds(0, _S)]
        hbr1 = hbr_scr[:, pl.ds(_S, _S)]
        tc1 = tcol_ref[pl.ds(_S, _S), :]
        tr0 = trow_ref[:, pl.ds(0, _S)]
        tr1 = trow_ref[:, pl.ds(_S, _S)]

        g10 = _ntb(fb1, fb0)
        cm10, rc10 = _block_updates(g10, hb1, hbr0, tc1, tr0, labels_row)
        g11 = _ntb(fb1, fb1)
        cm11, rc11 = _block_updates(g11, hb1, hbr1, tc1, tr1, labels_row)

        colmax0 = jnp.maximum(colmax_scr[:, pl.ds(0, _S)], cm10)
        colmax_all = jnp.concatenate([colmax0, cm11], axis=1)
        rcls_all = jnp.maximum(rcls_scr[...], jnp.maximum(rc10, rc11))

        t_row = trow_ref[...]
        cmask = lax.broadcasted_iota(jnp.int32, (_C, 1), 0) == t_row
        ccls_col = jnp.max(jnp.where(cmask, colmax_all, _NEG_INF),
                           axis=1, keepdims=True)
        eye = (lax.broadcasted_iota(jnp.int32, (_C, 1), 0) ==
               lax.broadcasted_iota(jnp.int32, (1, _C), 1)).astype(jnp.float32)
        rcls_fin = jnp.maximum(rcls_all, -3.0e38)
        rcls_col = _nt(eye, rcls_fin)
        half_max = jnp.maximum(ccls_col, rcls_col)
        cmax = jnp.sqrt(jnp.clip(2.0 * half_max, 1e-12, None))
        counts_col = jnp.sum(cmask.astype(jnp.float32), axis=1,
                             keepdims=True)
        contrib = jnp.where(counts_col >= 2.0, 1.0 / cmax, 0.0)
        intra = jnp.sum(contrib)

        centers = cen_scr[...] / jnp.maximum(counts_col, 1.0)
        cc = centers * centers
        csq_col = jnp.sum(cc, axis=1, keepdims=True)
        csq_row = _nt(ones_row, cc)
        gc = _nt(centers, centers)
        dc = jnp.sqrt(jnp.clip(csq_col + csq_row - 2.0 * gc, 1e-12, None))
        t_col = tcol_ref[...]
        onehot_nc = (t_col == labels_row).astype(jnp.float32)
        counts_row = lax.dot_general(jnp.ones((1, _N), jnp.float32), onehot_nc,
                                     (((1,), (0,)), ((), ())),
                                     preferred_element_type=jnp.float32)
        valid = (counts_col > 0.0) & (counts_row > 0.0) & (dc > 0.0)
        min_inter = jnp.min(jnp.where(valid, dc, _POS_INF))

        out_ref[0, 0] = _ALPHA * (_MARGIN - min_inter) + _BETA * intra


def kernel(features, targets):
    t_col = targets.reshape(_N, 1).astype(jnp.int32)
    t_row = targets.reshape(1, _N).astype(jnp.int32)
    out = pl.pallas_call(
        _body,
        grid=(_NF,),
        in_specs=[
            pl.BlockSpec((_F, _D), lambda s: (s, 0)),
            pl.BlockSpec((_N, 1), lambda s: (0, 0)),
            pl.BlockSpec((1, _N), lambda s: (0, 0)),
        ],
        out_specs=pl.BlockSpec(memory_space=pltpu.SMEM),
        out_shape=jax.ShapeDtypeStruct((1, 1), jnp.float32),
        scratch_shapes=[
            pltpu.VMEM((_N, _D), jnp.bfloat16),
            pltpu.VMEM((_N, 1), jnp.bfloat16),
            pltpu.VMEM((1, _N), jnp.bfloat16),
            pltpu.VMEM((_C, _D), jnp.float32),
            pltpu.VMEM((1, _N), jnp.float32),
            pltpu.VMEM((1, _C), jnp.float32),
        ],
    )(features, t_col, t_row)
    return out[0, 0]
